# baseline (speedup 1.0000x reference)
import functools

import jax
import jax.numpy as jnp
from jax.experimental import pallas as pl
from jax.experimental.pallas import tpu as pltpu

_BO = 4096
_BI = 2048
_PR = 512
_F8 = jnp.float8_e4m3fn
_F4 = jnp.float4_e2m1fn


def _prep_body(a_ref, x_ref, w1_ref, b1_ref, pa_ref, w2_ref, b2_ref,
               a8_ref, dinv_ref, h0_ref, acc_ref, *, n_panels, n_cblk, bo):
    r = pl.program_id(0)
    n = a_ref.shape[1]
    ch = min(2048, bo)
    csum = None
    for j in range(n // ch):
        aq = a_ref[:, j * ch:(j + 1) * ch]
        qb, lo = (j * ch) // bo, (j * ch) % bo
        a8_ref[0, qb, :, lo:lo + ch] = aq.astype(_F4)
        s = jnp.sum(aq, axis=0, keepdims=True)
        csum = s if csum is None else jnp.concatenate([csum, s], axis=1)

    t = jnp.dot(x_ref[...], w1_ref[...], preferred_element_type=jnp.float32)
    t = t + b1_ref[...]
    alpha = pa_ref[0, 0]
    t = jnp.where(t >= 0, t, alpha * t)
    h0 = jnp.dot(t, w2_ref[...], preferred_element_type=jnp.float32)
    h0_ref[...] = h0 + b2_ref[...]

    @pl.when(r == 0)
    def _():
        acc_ref[...] = csum

    @pl.when(r != 0)
    def _():
        acc_ref[...] += csum

    @pl.when(r == n_panels - 1)
    def _():
        deg = acc_ref[...] + 1.0
        dinv_ref[...] = 1.0 / jnp.sqrt(deg)


def _prep(A, x, W1, b1, prelu_a, W2, b2):
    n = A.shape[0]
    d = x.shape[1]
    c = W2.shape[1]
    pr, bo = min(_PR, n), min(_BO, n)
    n_panels, n_cblk = n // pr, n // bo
    return pl.pallas_call(
        functools.partial(_prep_body, n_panels=n_panels, n_cblk=n_cblk,
                          bo=bo),
        grid=(n_panels,),
        in_specs=[
            pl.BlockSpec((pr, n), lambda r: (r, 0)),
            pl.BlockSpec((pr, d), lambda r: (r, 0)),
            pl.BlockSpec((d, d), lambda r: (0, 0)),
            pl.BlockSpec((1, d), lambda r: (0, 0)),
            pl.BlockSpec(memory_space=pltpu.SMEM),
            pl.BlockSpec((d, c), lambda r: (0, 0)),
            pl.BlockSpec((1, c), lambda r: (0, 0)),
        ],
        out_specs=[
            pl.BlockSpec((1, n_cblk, pr, bo), lambda r: (r, 0, 0, 0)),
            pl.BlockSpec((1, n), lambda r: (0, 0)),
            pl.BlockSpec((pr, c), lambda r: (r, 0)),
        ],
        out_shape=[
            jax.ShapeDtypeStruct((n_panels, n_cblk, pr, bo), _F4),
            jax.ShapeDtypeStruct((1, n), jnp.float32),
            jax.ShapeDtypeStruct((n, c), jnp.float32),
        ],
        scratch_shapes=[pltpu.VMEM((1, n), jnp.float32)],
        compiler_params=pltpu.CompilerParams(
            dimension_semantics=("arbitrary",)
        ),
    )(A, x, W1, b1.reshape(1, d), prelu_a.reshape(1, 1), W2,
      b2.reshape(1, c))


def _hops_body(temp_ref, a8_ref, h0_ref, g_ref, out_ref,
               h_ref, hid_ref, u012_ref, acc_ref, s_ref,
               *, k_hops, n_out, n_in, bo, bi, pr, c):
    k = pl.program_id(0)
    o = pl.program_id(1)
    i = pl.program_id(2)

    n = h_ref.shape[0]
    ch = min(2048, n)

    @pl.when((k == 0) & (o == 0) & (i == 0))
    def _():
        for j in range(n // ch):
            h0 = h0_ref[pl.ds(j * ch, ch), :]
            h_ref[pl.ds(j * ch, ch), :] = h0
            hid_ref[pl.ds(j * ch, ch), :] = temp_ref[0] * h0

    @pl.when((o == 0) & (i == 0))
    def _():
        m = jnp.float32(1e-30)
        for j in range(n // ch):
            uj = g_ref[pl.ds(j * ch, ch), :] * h_ref[pl.ds(j * ch, ch), :]
            m = jnp.maximum(m, jnp.max(jnp.abs(uj)))
        s = jnp.exp2(jnp.ceil(jnp.log2(m)) - 7.0)
        s_ref[0, 0] = s
        inv = 1.0 / s
        for j in range(n // ch):
            sl = pl.ds(j * ch, ch)
            up = g_ref[sl, :] * h_ref[sl, :] * inv
            u0 = up.astype(_F8)
            r1 = (up - u0.astype(jnp.float32)) * 8.0
            u1 = r1.astype(_F8)
            r2 = (r1 - u1.astype(jnp.float32)) * 8.0
            u2 = r2.astype(_F8)
            u012_ref[sl, :] = jnp.concatenate([u0, u1, u2], axis=1)

    dn = (((0,), (0,)), ((), ()))
    npan = bi // pr
    for p in range(npan):
        lo = (i * npan + p) * pr
        u_sl = u012_ref[pl.ds(lo, pr), :]
        for ob in range(bo // ch):
            a = a8_ref[p, 0, :, ob * ch:(ob + 1) * ch]
            y = jax.lax.dot_general(a, u_sl, dn,
                                    preferred_element_type=jnp.float32)
            contrib = (y[:, :c] + y[:, c:2 * c] * 0.125
                       + y[:, 2 * c:] * 0.015625)
            osl = pl.ds(ob * ch, ch)
            if p == 0:
                @pl.when(i == 0)
                def _(contrib=contrib, osl=osl):
                    acc_ref[osl, :] = contrib

                @pl.when(i != 0)
                def _(contrib=contrib, osl=osl):
                    acc_ref[osl, :] += contrib
            else:
                acc_ref[osl, :] += contrib

    @pl.when(i == n_in - 1)
    def _():
        for j in range(bo // ch):
            row = o * bo + j * ch
            g_o = g_ref[pl.ds(row, ch), :]
            h_new = g_o * (s_ref[0, 0] * acc_ref[pl.ds(j * ch, ch), :]
                           + g_o * h_ref[pl.ds(row, ch), :])
            h_ref[pl.ds(row, ch), :] = h_new
            hid_new = hid_ref[pl.ds(row, ch), :] + temp_ref[k + 1] * h_new
            hid_ref[pl.ds(row, ch), :] = hid_new

            @pl.when(k == k_hops - 1)
            def _(hid_new=hid_new, row=row):
                m = jnp.max(hid_new, axis=1, keepdims=True)
                lse = m + jnp.log(jnp.sum(jnp.exp(hid_new - m), axis=1,
                                          keepdims=True))
                out_ref[pl.ds(row, ch), :] = hid_new - lse


def _hops(A8, h0, dinv_col, temp):
    n, c = h0.shape
    bo, bi, pr = min(_BO, n), min(_BI, n), min(_PR, n)
    k_hops = temp.shape[0] - 1
    n_out, n_in = n // bo, n // bi
    npan = bi // pr
    body = functools.partial(_hops_body, k_hops=k_hops, n_out=n_out,
                             n_in=n_in, bo=bo, bi=bi, pr=pr, c=c)
    return pl.pallas_call(
        body,
        grid=(k_hops, n_out, n_in),
        in_specs=[
            pl.BlockSpec(memory_space=pltpu.SMEM),
            pl.BlockSpec((npan, 1, pr, bo), lambda k, o, i: (i, o, 0, 0)),
            pl.BlockSpec((n, c), lambda k, o, i: (0, 0)),
            pl.BlockSpec((n, 1), lambda k, o, i: (0, 0)),
        ],
        out_specs=pl.BlockSpec((n, c), lambda k, o, i: (0, 0)),
        out_shape=jax.ShapeDtypeStruct((n, c), jnp.float32),
        scratch_shapes=[
            pltpu.VMEM((n, c), jnp.float32),
            pltpu.VMEM((n, c), jnp.float32),
            pltpu.VMEM((n, 3 * c), _F8),
            pltpu.VMEM((bo, c), jnp.float32),
            pltpu.SMEM((1, 1), jnp.float32),
        ],
        compiler_params=pltpu.CompilerParams(
            dimension_semantics=("arbitrary", "arbitrary", "arbitrary")
        ),
    )(temp, A8, h0, dinv_col)


def kernel(x, A, W1, b1, prelu_a, W2, b2, temp):
    n = A.shape[0]
    A8, dinv, h0 = _prep(A, x, W1, b1, prelu_a, W2, b2)
    dinv_col = dinv.reshape(n, 1)
    return _hops(A8, h0, dinv_col, temp)


# speedup vs baseline: 1.0451x; 1.0451x over previous
import functools

import jax
import jax.numpy as jnp
from jax.experimental import pallas as pl
from jax.experimental.pallas import tpu as pltpu

_BO = 4096
_BI = 2048
_PR = 512
_F8 = jnp.float8_e4m3fn
_F4 = jnp.float4_e2m1fn


def _prep_body(a_ref, x_ref, w1_ref, b1_ref, pa_ref, w2_ref, b2_ref,
               a8_ref, dinv_ref, h0_ref, acc_ref, *, n_panels, n_cblk, bo):
    r = pl.program_id(0)
    n = a_ref.shape[1]
    ch = min(2048, bo)
    csum = None
    for j in range(n // ch):
        aq = a_ref[:, j * ch:(j + 1) * ch]
        qb, lo = (j * ch) // bo, (j * ch) % bo
        a8_ref[0, qb, :, lo:lo + ch] = aq.astype(_F4)
        s = jnp.sum(aq, axis=0, keepdims=True)
        csum = s if csum is None else jnp.concatenate([csum, s], axis=1)

    t = jnp.dot(x_ref[...], w1_ref[...], preferred_element_type=jnp.float32)
    t = t + b1_ref[...]
    alpha = pa_ref[0, 0]
    t = jnp.where(t >= 0, t, alpha * t)
    h0 = jnp.dot(t, w2_ref[...], preferred_element_type=jnp.float32)
    h0_ref[...] = h0 + b2_ref[...]

    @pl.when(r == 0)
    def _():
        acc_ref[...] = csum

    @pl.when(r != 0)
    def _():
        acc_ref[...] += csum

    @pl.when(r == n_panels - 1)
    def _():
        deg = acc_ref[...] + 1.0
        dinv_ref[...] = 1.0 / jnp.sqrt(deg)


def _prep(A, x, W1, b1, prelu_a, W2, b2):
    n = A.shape[0]
    d = x.shape[1]
    c = W2.shape[1]
    pr, bo = min(_PR, n), min(_BO, n)
    n_panels, n_cblk = n // pr, n // bo
    return pl.pallas_call(
        functools.partial(_prep_body, n_panels=n_panels, n_cblk=n_cblk,
                          bo=bo),
        grid=(n_panels,),
        in_specs=[
            pl.BlockSpec((pr, n), lambda r: (r, 0)),
            pl.BlockSpec((pr, d), lambda r: (r, 0)),
            pl.BlockSpec((d, d), lambda r: (0, 0)),
            pl.BlockSpec((1, d), lambda r: (0, 0)),
            pl.BlockSpec(memory_space=pltpu.SMEM),
            pl.BlockSpec((d, c), lambda r: (0, 0)),
            pl.BlockSpec((1, c), lambda r: (0, 0)),
        ],
        out_specs=[
            pl.BlockSpec((1, n_cblk, pr, bo), lambda r: (r, 0, 0, 0)),
            pl.BlockSpec((1, n), lambda r: (0, 0)),
            pl.BlockSpec((pr, c), lambda r: (r, 0)),
        ],
        out_shape=[
            jax.ShapeDtypeStruct((n_panels, n_cblk, pr, bo), _F4),
            jax.ShapeDtypeStruct((1, n), jnp.float32),
            jax.ShapeDtypeStruct((n, c), jnp.float32),
        ],
        scratch_shapes=[pltpu.VMEM((1, n), jnp.float32)],
        compiler_params=pltpu.CompilerParams(
            dimension_semantics=("arbitrary",)
        ),
    )(A, x, W1, b1.reshape(1, d), prelu_a.reshape(1, 1), W2,
      b2.reshape(1, c))


def _hops_body(temp_ref, a8_ref, h0_ref, g_ref, out_ref,
               h_ref, hid_ref, u012_ref, acc_ref, s_ref,
               *, k_hops, n_out, n_in, bo, bi, pr, c):
    k = pl.program_id(0)
    o = pl.program_id(1)
    i = pl.program_id(2)

    @pl.when((k == 0) & (o == 0) & (i == 0))
    def _():
        h0 = h0_ref[...]
        h_ref[...] = h0
        hid_ref[...] = temp_ref[0] * h0

    @pl.when((o == 0) & (i == 0))
    def _():
        u = g_ref[...] * h_ref[...]
        m = jnp.max(jnp.abs(u))
        m = jnp.maximum(m, 1e-30)
        s = jnp.exp2(jnp.ceil(jnp.log2(m)) - 7.0)
        s_ref[0, 0] = s
        up = u * (1.0 / s)
        u0 = up.astype(_F8)
        r1 = (up - u0.astype(jnp.float32)) * 8.0
        u1 = r1.astype(_F8)
        r2 = (r1 - u1.astype(jnp.float32)) * 8.0
        u2 = r2.astype(_F8)
        u012_ref[...] = jnp.concatenate([u0, u1, u2], axis=1)

    dn = (((0,), (0,)), ((), ()))
    npan = bi // pr
    part = None
    for p in range(npan):
        a = a8_ref[p, 0]
        lo = (i * npan + p) * pr
        y = jax.lax.dot_general(
            a, u012_ref[pl.ds(lo, pr), :], dn,
            preferred_element_type=jnp.float32)
        contrib = (y[:, :c] + y[:, c:2 * c] * 0.125
                   + y[:, 2 * c:] * 0.015625)
        part = contrib if part is None else part + contrib

    @pl.when(i == 0)
    def _():
        acc_ref[...] = part

    @pl.when(i != 0)
    def _():
        acc_ref[...] += part

    @pl.when(i == n_in - 1)
    def _():
        g_o = g_ref[pl.ds(o * bo, bo), :]
        h_new = g_o * (s_ref[0, 0] * acc_ref[...]
                       + g_o * h_ref[pl.ds(o * bo, bo), :])
        h_ref[pl.ds(o * bo, bo), :] = h_new
        hid_new = hid_ref[pl.ds(o * bo, bo), :] + temp_ref[k + 1] * h_new
        hid_ref[pl.ds(o * bo, bo), :] = hid_new

        @pl.when(k == k_hops - 1)
        def _():
            m = jnp.max(hid_new, axis=1, keepdims=True)
            lse = m + jnp.log(jnp.sum(jnp.exp(hid_new - m), axis=1,
                                      keepdims=True))
            out_ref[pl.ds(o * bo, bo), :] = hid_new - lse


def _hops(A8, h0, dinv_col, temp):
    n, c = h0.shape
    bo, bi, pr = min(_BO, n), min(_BI, n), min(_PR, n)
    k_hops = temp.shape[0] - 1
    n_out, n_in = n // bo, n // bi
    npan = bi // pr
    body = functools.partial(_hops_body, k_hops=k_hops, n_out=n_out,
                             n_in=n_in, bo=bo, bi=bi, pr=pr, c=c)
    return pl.pallas_call(
        body,
        grid=(k_hops, n_out, n_in),
        in_specs=[
            pl.BlockSpec(memory_space=pltpu.SMEM),
            pl.BlockSpec((npan, 1, pr, bo), lambda k, o, i: (i, o, 0, 0)),
            pl.BlockSpec((n, c), lambda k, o, i: (0, 0)),
            pl.BlockSpec((n, 1), lambda k, o, i: (0, 0)),
        ],
        out_specs=pl.BlockSpec((n, c), lambda k, o, i: (0, 0)),
        out_shape=jax.ShapeDtypeStruct((n, c), jnp.float32),
        scratch_shapes=[
            pltpu.VMEM((n, c), jnp.float32),
            pltpu.VMEM((n, c), jnp.float32),
            pltpu.VMEM((n, 3 * c), _F8),
            pltpu.VMEM((bo, c), jnp.float32),
            pltpu.SMEM((1, 1), jnp.float32),
        ],
        compiler_params=pltpu.CompilerParams(
            dimension_semantics=("arbitrary", "arbitrary", "arbitrary")
        ),
    )(temp, A8, h0, dinv_col)


def kernel(x, A, W1, b1, prelu_a, W2, b2, temp):
    n = A.shape[0]
    A8, dinv, h0 = _prep(A, x, W1, b1, prelu_a, W2, b2)
    dinv_col = dinv.reshape(n, 1)
    return _hops(A8, h0, dinv_col, temp)
